# baseline (speedup 1.0000x reference)
import jax
import jax.numpy as jnp
from jax.experimental import pallas as pl
from jax.experimental.pallas import tpu as pltpu

_BLOCK = 512
_P = (1024, 896, 512, 640, 128)


def _mlp_kernel(x_ref, w1_ref, b1_ref, w2_ref, b2_ref, w3_ref, b3_ref,
                w4_ref, b4_ref, w5_ref, b5_ref,
                o1, o2, o3, o4, o5,
                s1, s2, s3, s4, s5, sems):
    i = pl.program_id(0)
    nsteps = pl.num_programs(0)
    blk = x_ref.shape[0]
    dn = (((1,), (1,)), ((), ()))
    bf = jnp.bfloat16

    x = x_ref[...].astype(bf)
    h1 = jax.lax.dot_general(x, w1_ref[...].astype(bf), dn,
                             preferred_element_type=jnp.float32)
    h1 = jnp.maximum(h1 + b1_ref[...], 0.0)
    h2 = jax.lax.dot_general(h1.astype(bf), w2_ref[...].astype(bf), dn,
                             preferred_element_type=jnp.float32)
    h2 = jnp.maximum(h2 + b2_ref[...], 0.0)
    h3 = jax.lax.dot_general(h2.astype(bf), w3_ref[...].astype(bf), dn,
                             preferred_element_type=jnp.float32)
    h3 = jnp.maximum(h3 + b3_ref[...], 0.0)
    h4 = jax.lax.dot_general(h3.astype(bf), w4_ref[...].astype(bf), dn,
                             preferred_element_type=jnp.float32)
    h4 = h4 + b4_ref[...]
    h5 = jax.lax.dot_general(h4.astype(bf), w5_ref[...].astype(bf), dn,
                             preferred_element_type=jnp.float32)
    h5 = h5 + b5_ref[...]

    outs = ((s1, o1, h1), (s2, o2, h2), (s3, o3, h3), (s4, o4, h4),
            (s5, o5, h5))

    def issue(slot):
        for k, (scr, hbm, hval) in enumerate(outs):
            d = hbm.shape[1]
            cp = pltpu.make_async_copy(
                scr.at[slot], hbm.at[pl.ds(i * blk, blk), :],
                sems.at[k, slot])

            @pl.when(i >= 2)
            def _wait():
                cp.wait()

            scr[slot] = hval[:, :d]
            cp.start()

    @pl.when(i % 2 == 0)
    def _even():
        issue(0)

    @pl.when(i % 2 == 1)
    def _odd():
        issue(1)

    @pl.when(i == nsteps - 1)
    def _drain():
        for k, (scr, hbm, _) in enumerate(outs):
            for slot in (0, 1):
                pltpu.make_async_copy(
                    scr.at[slot], hbm.at[pl.ds(i * blk, blk), :],
                    sems.at[k, slot]).wait()


def kernel(x, W1, b1, M1, W2, b2, M2, W3, b3, M3, W4, b4, M4, W5, b5, M5):
    n, d_in = x.shape
    d1, d2, d3, d4, d5 = (W1.shape[0], W2.shape[0], W3.shape[0],
                          W4.shape[0], W5.shape[0])
    p1, p2, p3, p4, p5 = _P

    def _pad_to(a, rows, cols):
        return jnp.pad(a, ((0, rows - a.shape[0]), (0, cols - a.shape[1])))

    W1p = _pad_to(W1 * M1, p1, d_in)
    W2p = _pad_to(W2 * M2, p2, p1)
    W3p = _pad_to(W3 * M3, p3, p2)
    W4p = _pad_to(W4 * M4, p4, p3)
    W5p = _pad_to(W5 * M5, p5, p4)
    b1p = jnp.pad(b1, (0, p1 - d1)).reshape(1, -1)
    b2p = jnp.pad(b2, (0, p2 - d2)).reshape(1, -1)
    b3p = jnp.pad(b3, (0, p3 - d3)).reshape(1, -1)
    b4p = jnp.pad(b4, (0, p4 - d4)).reshape(1, -1)
    b5p = jnp.pad(b5, (0, p5 - d5)).reshape(1, -1)

    def wspec(a):
        return pl.BlockSpec(a.shape, lambda i: (0, 0))

    block = _BLOCK
    hbm_spec = pl.BlockSpec(memory_space=pltpu.MemorySpace.HBM)
    h1, h2, h3, h4, h5 = pl.pallas_call(
        _mlp_kernel,
        grid=(n // block,),
        in_specs=[
            pl.BlockSpec((block, d_in), lambda i: (i, 0)),
            wspec(W1p), wspec(b1p),
            wspec(W2p), wspec(b2p),
            wspec(W3p), wspec(b3p),
            wspec(W4p), wspec(b4p),
            wspec(W5p), wspec(b5p),
        ],
        out_specs=[hbm_spec, hbm_spec, hbm_spec, hbm_spec, hbm_spec],
        out_shape=[
            jax.ShapeDtypeStruct((n, d1), jnp.float32),
            jax.ShapeDtypeStruct((n, d2), jnp.float32),
            jax.ShapeDtypeStruct((n, d3), jnp.float32),
            jax.ShapeDtypeStruct((n, d4), jnp.float32),
            jax.ShapeDtypeStruct((n, d5), jnp.float32),
        ],
        scratch_shapes=[
            pltpu.VMEM((2, block, d1), jnp.float32),
            pltpu.VMEM((2, block, d2), jnp.float32),
            pltpu.VMEM((2, block, d3), jnp.float32),
            pltpu.VMEM((2, block, d4), jnp.float32),
            pltpu.VMEM((2, block, d5), jnp.float32),
            pltpu.SemaphoreType.DMA((5, 2)),
        ],
        compiler_params=pltpu.CompilerParams(
            dimension_semantics=("arbitrary",),
        ),
    )(x, W1p, b1p, W2p, b2p, W3p, b3p, W4p, b4p, W5p, b5p)
    return (h5, h1, h2, h3, h4, h5)


# speedup vs baseline: 1.1022x; 1.1022x over previous
import jax
import jax.numpy as jnp
from jax.experimental import pallas as pl
from jax.experimental.pallas import tpu as pltpu

_BLOCK = 1024


def _mlp_kernel(x_ref, w1_ref, b1_ref, m1_ref, w2_ref, b2_ref, m2_ref,
                w3_ref, b3_ref, m3_ref, w4_ref, b4_ref, m4_ref,
                w5_ref, b5_ref, m5_ref,
                h1_ref, h2_ref, h3_ref, h4_ref, h5_ref):
    dn = (((1,), (1,)), ((), ()))

    x = x_ref[...]
    w1 = w1_ref[...] * m1_ref[...]
    h1 = jax.lax.dot_general(x, w1, dn, preferred_element_type=jnp.float32)
    h1 = jnp.maximum(h1 + b1_ref[...], 0.0)
    h1_ref[...] = h1

    w2 = w2_ref[...] * m2_ref[...]
    h2 = jax.lax.dot_general(h1, w2, dn, preferred_element_type=jnp.float32)
    h2 = jnp.maximum(h2 + b2_ref[...], 0.0)
    h2_ref[...] = h2

    w3 = w3_ref[...] * m3_ref[...]
    h3 = jax.lax.dot_general(h2, w3, dn, preferred_element_type=jnp.float32)
    h3 = jnp.maximum(h3 + b3_ref[...], 0.0)
    h3_ref[...] = h3

    w4 = w4_ref[...] * m4_ref[...]
    h4 = jax.lax.dot_general(h3, w4, dn, preferred_element_type=jnp.float32)
    h4 = h4 + b4_ref[...]
    h4_ref[...] = h4

    w5 = w5_ref[...] * m5_ref[...]
    h5 = jax.lax.dot_general(h4, w5, dn, preferred_element_type=jnp.float32)
    h5 = h5 + b5_ref[...]
    h5_ref[...] = h5


def kernel(x, W1, b1, M1, W2, b2, M2, W3, b3, M3, W4, b4, M4, W5, b5, M5):
    n, d_in = x.shape
    d1, d2, d3, d4, d5 = (W1.shape[0], W2.shape[0], W3.shape[0],
                          W4.shape[0], W5.shape[0])
    b1, b2, b3, b4, b5 = (b.reshape(1, -1) for b in (b1, b2, b3, b4, b5))

    def wspec(a):
        return pl.BlockSpec(a.shape, lambda i: (0, 0))

    block = _BLOCK
    h1, h2, h3, h4, h5 = pl.pallas_call(
        _mlp_kernel,
        grid=(n // block,),
        in_specs=[
            pl.BlockSpec((block, d_in), lambda i: (i, 0)),
            wspec(W1), wspec(b1), wspec(M1),
            wspec(W2), wspec(b2), wspec(M2),
            wspec(W3), wspec(b3), wspec(M3),
            wspec(W4), wspec(b4), wspec(M4),
            wspec(W5), wspec(b5), wspec(M5),
        ],
        out_specs=[
            pl.BlockSpec((block, d1), lambda i: (i, 0)),
            pl.BlockSpec((block, d2), lambda i: (i, 0)),
            pl.BlockSpec((block, d3), lambda i: (i, 0)),
            pl.BlockSpec((block, d4), lambda i: (i, 0)),
            pl.BlockSpec((block, d5), lambda i: (i, 0)),
        ],
        out_shape=[
            jax.ShapeDtypeStruct((n, d1), jnp.float32),
            jax.ShapeDtypeStruct((n, d2), jnp.float32),
            jax.ShapeDtypeStruct((n, d3), jnp.float32),
            jax.ShapeDtypeStruct((n, d4), jnp.float32),
            jax.ShapeDtypeStruct((n, d5), jnp.float32),
        ],
        compiler_params=pltpu.CompilerParams(
            dimension_semantics=("arbitrary",),
        ),
    )(x, W1, b1, M1, W2, b2, M2, W3, b3, M3, W4, b4, M4, W5, b5, M5)
    return (h5, h1, h2, h3, h4, h5)


# speedup vs baseline: 1.2338x; 1.1194x over previous
import jax
import jax.numpy as jnp
from jax.experimental import pallas as pl
from jax.experimental.pallas import tpu as pltpu

_BLOCK = 1024


def _mlp_kernel(x_ref, w1_ref, b1_ref, m1_ref, w2_ref, b2_ref, m2_ref,
                w3_ref, b3_ref, m3_ref, w4_ref, b4_ref, m4_ref,
                w5_ref, b5_ref, m5_ref,
                h1_ref, h2_ref, h3_ref, h4_ref, h5_ref):
    dn = (((1,), (1,)), ((), ()))

    x = x_ref[...]
    w1 = w1_ref[...] * m1_ref[...]
    h1 = jax.lax.dot_general(x, w1, dn, preferred_element_type=jnp.float32)
    h1 = jnp.maximum(h1 + b1_ref[...], 0.0)
    h1_ref[...] = h1.astype(jnp.bfloat16)

    w2 = w2_ref[...] * m2_ref[...]
    h2 = jax.lax.dot_general(h1, w2, dn, preferred_element_type=jnp.float32)
    h2 = jnp.maximum(h2 + b2_ref[...], 0.0)
    h2_ref[...] = h2.astype(jnp.bfloat16)

    w3 = w3_ref[...] * m3_ref[...]
    h3 = jax.lax.dot_general(h2, w3, dn, preferred_element_type=jnp.float32)
    h3 = jnp.maximum(h3 + b3_ref[...], 0.0)
    h3_ref[...] = h3.astype(jnp.bfloat16)

    w4 = w4_ref[...] * m4_ref[...]
    h4 = jax.lax.dot_general(h3, w4, dn, preferred_element_type=jnp.float32)
    h4 = h4 + b4_ref[...]
    h4_ref[...] = h4.astype(jnp.bfloat16)

    w5 = w5_ref[...] * m5_ref[...]
    h5 = jax.lax.dot_general(h4, w5, dn, preferred_element_type=jnp.float32)
    h5 = h5 + b5_ref[...]
    h5_ref[...] = h5.astype(jnp.bfloat16)


def kernel(x, W1, b1, M1, W2, b2, M2, W3, b3, M3, W4, b4, M4, W5, b5, M5):
    n, d_in = x.shape
    d1, d2, d3, d4, d5 = (W1.shape[0], W2.shape[0], W3.shape[0],
                          W4.shape[0], W5.shape[0])
    b1, b2, b3, b4, b5 = (b.reshape(1, -1) for b in (b1, b2, b3, b4, b5))

    def wspec(a):
        return pl.BlockSpec(a.shape, lambda i: (0, 0))

    block = _BLOCK
    h1, h2, h3, h4, h5 = pl.pallas_call(
        _mlp_kernel,
        grid=(n // block,),
        in_specs=[
            pl.BlockSpec((block, d_in), lambda i: (i, 0)),
            wspec(W1), wspec(b1), wspec(M1),
            wspec(W2), wspec(b2), wspec(M2),
            wspec(W3), wspec(b3), wspec(M3),
            wspec(W4), wspec(b4), wspec(M4),
            wspec(W5), wspec(b5), wspec(M5),
        ],
        out_specs=[
            pl.BlockSpec((block, d1), lambda i: (i, 0)),
            pl.BlockSpec((block, d2), lambda i: (i, 0)),
            pl.BlockSpec((block, d3), lambda i: (i, 0)),
            pl.BlockSpec((block, d4), lambda i: (i, 0)),
            pl.BlockSpec((block, d5), lambda i: (i, 0)),
        ],
        out_shape=[
            jax.ShapeDtypeStruct((n, d1), jnp.bfloat16),
            jax.ShapeDtypeStruct((n, d2), jnp.bfloat16),
            jax.ShapeDtypeStruct((n, d3), jnp.bfloat16),
            jax.ShapeDtypeStruct((n, d4), jnp.bfloat16),
            jax.ShapeDtypeStruct((n, d5), jnp.bfloat16),
        ],
        compiler_params=pltpu.CompilerParams(
            dimension_semantics=("arbitrary",),
        ),
    )(x, W1, b1, M1, W2, b2, M2, W3, b3, M3, W4, b4, M4, W5, b5, M5)
    h1 = h1.astype(jnp.float32)
    h2 = h2.astype(jnp.float32)
    h3 = h3.astype(jnp.float32)
    h4 = h4.astype(jnp.float32)
    h5 = h5.astype(jnp.float32)
    return (h5, h1, h2, h3, h4, h5)
